# initial kernel scaffold (unmeasured)
import jax
import jax.numpy as jnp
from jax import lax
from jax.experimental import pallas as pl
from jax.experimental.pallas import tpu as pltpu

N_DEV = 32


def kernel(x, w_mat, scale_x, scale_w):
    m_per, k = x.shape
    _, n_per = w_mat.shape
    m_tot = N_DEV * m_per

    scale = (scale_x * scale_w).reshape(1, 1)

    def body(x_ref, w_ref, s_ref, out_ref, xg_ref, copy_sem, send_sems, recv_sems):
        my = lax.axis_index("i")
        left = lax.rem(my + N_DEV - 1, N_DEV)
        right = lax.rem(my + 1, N_DEV)

        barrier_sem = pltpu.get_barrier_semaphore()
        for nbr in (left, right):
            pl.semaphore_signal(
                barrier_sem, inc=1, device_id=(nbr,),
                device_id_type=pl.DeviceIdType.MESH,
            )
        pl.semaphore_wait(barrier_sem, 2)

        lcopy = pltpu.make_async_copy(x_ref, xg_ref.at[my], copy_sem)
        lcopy.start()
        lcopy.wait()

        for h in range(N_DEV - 1):
            slot = lax.rem(my - h + N_DEV, N_DEV)
            rdma = pltpu.make_async_remote_copy(
                src_ref=xg_ref.at[slot],
                dst_ref=xg_ref.at[slot],
                send_sem=send_sems.at[h],
                recv_sem=recv_sems.at[h],
                device_id=(right,),
                device_id_type=pl.DeviceIdType.MESH,
            )
            rdma.start()
            rdma.wait()

        s = s_ref[0, 0]
        w = w_ref[...]
        for o in range(N_DEV):
            acc = jnp.dot(xg_ref[o], w, preferred_element_type=jnp.float32)
            y = acc * s
            out_ref[o * m_per:(o + 1) * m_per, :] = y * jax.nn.sigmoid(y)

    return pl.pallas_call(
        body,
        out_shape=jax.ShapeDtypeStruct((m_tot, n_per), jnp.float32),
        in_specs=[
            pl.BlockSpec(memory_space=pltpu.VMEM),
            pl.BlockSpec(memory_space=pltpu.VMEM),
            pl.BlockSpec(memory_space=pltpu.SMEM),
        ],
        out_specs=pl.BlockSpec(memory_space=pltpu.VMEM),
        scratch_shapes=[
            pltpu.VMEM((N_DEV, m_per, k), x.dtype),
            pltpu.SemaphoreType.DMA,
            pltpu.SemaphoreType.DMA((N_DEV - 1,)),
            pltpu.SemaphoreType.DMA((N_DEV - 1,)),
        ],
        compiler_params=pltpu.CompilerParams(collective_id=0),
    )(x, w_mat, scale)


# baseline (device time: 244275 ns/iter reference)
import jax
import jax.numpy as jnp
from jax import lax
from jax.experimental import pallas as pl
from jax.experimental.pallas import tpu as pltpu

N_DEV = 32


def kernel(x, w_mat, scale_x, scale_w):
    m_per, k = x.shape
    _, n_per = w_mat.shape
    m_tot = N_DEV * m_per

    scale = (scale_x * scale_w).reshape(1, 1)
    x = x.astype(jnp.float8_e5m2)
    w_mat = w_mat.astype(jnp.float8_e5m2)

    def body(x_ref, w_ref, s_ref, out_ref, xg_ref, copy_sem, send_sems, recv_sems):
        my = lax.axis_index("i")
        left = lax.rem(my + N_DEV - 1, N_DEV)
        right = lax.rem(my + 1, N_DEV)

        barrier_sem = pltpu.get_barrier_semaphore()
        for nbr in (left, right):
            pl.semaphore_signal(
                barrier_sem, inc=1, device_id=(nbr,),
                device_id_type=pl.DeviceIdType.MESH,
            )
        pl.semaphore_wait(barrier_sem, 2)

        lcopy = pltpu.make_async_copy(x_ref, xg_ref.at[my], copy_sem)
        lcopy.start()
        lcopy.wait()

        for h in range(N_DEV - 1):
            slot = lax.rem(my - h + N_DEV, N_DEV)
            rdma = pltpu.make_async_remote_copy(
                src_ref=xg_ref.at[slot],
                dst_ref=xg_ref.at[slot],
                send_sem=send_sems.at[h],
                recv_sem=recv_sems.at[h],
                device_id=(right,),
                device_id_type=pl.DeviceIdType.MESH,
            )
            rdma.start()
            rdma.wait()

        s = s_ref[0, 0]
        w = w_ref[...]
        for o in range(N_DEV):
            acc = jnp.dot(xg_ref[o], w, preferred_element_type=jnp.float32)
            y = acc * s
            out_ref[o * m_per:(o + 1) * m_per, :] = y * jax.nn.sigmoid(y)

    return pl.pallas_call(
        body,
        out_shape=jax.ShapeDtypeStruct((m_tot, n_per), jnp.float32),
        in_specs=[
            pl.BlockSpec(memory_space=pltpu.VMEM),
            pl.BlockSpec(memory_space=pltpu.VMEM),
            pl.BlockSpec(memory_space=pltpu.SMEM),
        ],
        out_specs=pl.BlockSpec(memory_space=pltpu.VMEM),
        scratch_shapes=[
            pltpu.VMEM((N_DEV, m_per, k), x.dtype),
            pltpu.SemaphoreType.DMA,
            pltpu.SemaphoreType.DMA((N_DEV - 1,)),
            pltpu.SemaphoreType.DMA((N_DEV - 1,)),
        ],
        compiler_params=pltpu.CompilerParams(collective_id=0),
    )(x, w_mat, scale)


# device time: 196676 ns/iter; 1.2420x vs baseline; 1.2420x over previous
import jax
import jax.numpy as jnp
from jax import lax
from jax.experimental import pallas as pl
from jax.experimental.pallas import tpu as pltpu

N_DEV = 32
N_HOPS = N_DEV - 1
SEG = 4


def kernel(x, w_mat, scale_x, scale_w):
    m_per, k = x.shape
    _, n_per = w_mat.shape
    m_tot = N_DEV * m_per
    half = m_per // 2
    seg_k = k // SEG

    scale = (scale_x * scale_w).reshape(1, 1)
    x = x.astype(jnp.float8_e5m2)
    w_mat = w_mat.astype(jnp.float8_e5m2)

    def body(x_ref, w_ref, s_ref, out_ref, xg_ref, copy_sem,
             send_cw, recv_cw, send_ccw, recv_ccw):
        my = lax.axis_index("i")
        left = lax.rem(my + N_DEV - 1, N_DEV)
        right = lax.rem(my + 1, N_DEV)

        barrier_sem = pltpu.get_barrier_semaphore()
        for nbr in (left, right):
            pl.semaphore_signal(
                barrier_sem, inc=1, device_id=(nbr,),
                device_id_type=pl.DeviceIdType.MESH,
            )
        pl.semaphore_wait(barrier_sem, 2)

        lcopy = pltpu.make_async_copy(x_ref, xg_ref.at[my], copy_sem)
        lcopy.start()
        lcopy.wait()

        def cw_desc(h, s, slot):
            blk = xg_ref.at[slot, pl.ds(0, half), pl.ds(s * seg_k, seg_k)]
            return pltpu.make_async_remote_copy(
                src_ref=blk, dst_ref=blk,
                send_sem=send_cw.at[h, s], recv_sem=recv_cw.at[h, s],
                device_id=(right,), device_id_type=pl.DeviceIdType.MESH,
            )

        def ccw_desc(h, s, slot):
            blk = xg_ref.at[slot, pl.ds(half, half), pl.ds(s * seg_k, seg_k)]
            return pltpu.make_async_remote_copy(
                src_ref=blk, dst_ref=blk,
                send_sem=send_ccw.at[h, s], recv_sem=recv_ccw.at[h, s],
                device_id=(left,), device_id_type=pl.DeviceIdType.MESH,
            )

        def cw_slot(h):
            return lax.rem(my - h + 2 * N_DEV, N_DEV)

        def ccw_slot(h):
            return lax.rem(my + h, N_DEV)

        for s in range(SEG):
            cw_desc(0, s, cw_slot(0)).start()
            ccw_desc(0, s, ccw_slot(0)).start()

        for h in range(N_HOPS):
            rs_cw = cw_slot(h + 1)
            rs_ccw = ccw_slot(h + 1)
            for s in range(SEG):
                cw_desc(h, s, rs_cw).wait_recv()
                if h + 1 < N_HOPS:
                    cw_desc(h + 1, s, rs_cw).start()
                ccw_desc(h, s, rs_ccw).wait_recv()
                if h + 1 < N_HOPS:
                    ccw_desc(h + 1, s, rs_ccw).start()

        for h in range(N_HOPS):
            for s in range(SEG):
                cw_desc(h, s, cw_slot(h)).wait_send()
                ccw_desc(h, s, ccw_slot(h)).wait_send()

        sc = s_ref[0, 0]
        w = w_ref[...]
        for o in range(N_DEV):
            acc = jnp.dot(xg_ref[o], w, preferred_element_type=jnp.float32)
            y = acc * sc
            out_ref[o * m_per:(o + 1) * m_per, :] = y * jax.nn.sigmoid(y)

    return pl.pallas_call(
        body,
        out_shape=jax.ShapeDtypeStruct((m_tot, n_per), jnp.float32),
        in_specs=[
            pl.BlockSpec(memory_space=pltpu.VMEM),
            pl.BlockSpec(memory_space=pltpu.VMEM),
            pl.BlockSpec(memory_space=pltpu.SMEM),
        ],
        out_specs=pl.BlockSpec(memory_space=pltpu.VMEM),
        scratch_shapes=[
            pltpu.VMEM((N_DEV, m_per, k), x.dtype),
            pltpu.SemaphoreType.DMA,
            pltpu.SemaphoreType.DMA((N_HOPS, SEG)),
            pltpu.SemaphoreType.DMA((N_HOPS, SEG)),
            pltpu.SemaphoreType.DMA((N_HOPS, SEG)),
            pltpu.SemaphoreType.DMA((N_HOPS, SEG)),
        ],
        compiler_params=pltpu.CompilerParams(collective_id=0),
    )(x, w_mat, scale)


# device time: 107396 ns/iter; 2.2745x vs baseline; 1.8313x over previous
import numpy as np

import jax
import jax.numpy as jnp
from jax import lax
from jax.experimental import pallas as pl
from jax.experimental.pallas import tpu as pltpu

N_DEV = 32
N_HOPS = N_DEV - 1
SEG = 4


def _ring_tables():
    def lid(x, y, z):
        return z * 8 + y * 2 + (x if y % 2 == 0 else 1 - x)

    c16 = [(0, 0), (1, 0), (2, 0), (3, 0), (3, 1), (2, 1), (1, 1), (1, 2),
           (2, 2), (3, 2), (3, 3), (2, 3), (1, 3), (0, 3), (0, 2), (0, 1)]
    ham = [lid(0, y, z) for (y, z) in c16] + \
          [lid(1, y, z) for (y, z) in reversed(c16)]
    pos = {l: p for p, l in enumerate(ham)}
    rn = np.array([ham[(pos[l] + 1) % N_DEV] for l in range(N_DEV)], np.int32)
    ln = np.array([ham[(pos[l] - 1) % N_DEV] for l in range(N_DEV)], np.int32)
    cw_t = np.array(
        [[ham[(pos[l] - h) % N_DEV] for l in range(N_DEV)]
         for h in range(N_DEV)], np.int32)
    ccw_t = np.array(
        [[ham[(pos[l] + h) % N_DEV] for l in range(N_DEV)]
         for h in range(N_DEV)], np.int32)
    return rn, ln, cw_t, ccw_t


_RN, _LN, _CW_T, _CCW_T = _ring_tables()


def kernel(x, w_mat, scale_x, scale_w):
    m_per, k = x.shape
    _, n_per = w_mat.shape
    m_tot = N_DEV * m_per
    half = m_per // 2
    seg_k = k // SEG

    scale = (scale_x * scale_w).reshape(1, 1)
    x = x.astype(jnp.float8_e5m2)
    w_mat = w_mat.astype(jnp.float8_e5m2)

    def body(x_ref, w_ref, s_ref, rn_ref, ln_ref, cwt_ref, ccwt_ref,
             out_ref, xg_ref, copy_sem, send_cw, recv_cw, send_ccw, recv_ccw):
        my = lax.axis_index("i")
        rn = rn_ref[my]
        ln = ln_ref[my]

        barrier_sem = pltpu.get_barrier_semaphore()
        for nbr in (ln, rn):
            pl.semaphore_signal(
                barrier_sem, inc=1, device_id=(nbr,),
                device_id_type=pl.DeviceIdType.MESH,
            )
        pl.semaphore_wait(barrier_sem, 2)

        lcopy = pltpu.make_async_copy(x_ref, xg_ref.at[my], copy_sem)
        lcopy.start()
        lcopy.wait()

        def cw_desc(h, s, slot):
            blk = xg_ref.at[slot, pl.ds(0, half), pl.ds(s * seg_k, seg_k)]
            return pltpu.make_async_remote_copy(
                src_ref=blk, dst_ref=blk,
                send_sem=send_cw.at[h, s], recv_sem=recv_cw.at[h, s],
                device_id=(rn,), device_id_type=pl.DeviceIdType.MESH,
            )

        def ccw_desc(h, s, slot):
            blk = xg_ref.at[slot, pl.ds(half, half), pl.ds(s * seg_k, seg_k)]
            return pltpu.make_async_remote_copy(
                src_ref=blk, dst_ref=blk,
                send_sem=send_ccw.at[h, s], recv_sem=recv_ccw.at[h, s],
                device_id=(ln,), device_id_type=pl.DeviceIdType.MESH,
            )

        for s in range(SEG):
            cw_desc(0, s, my).start()
            ccw_desc(0, s, my).start()

        for h in range(N_HOPS):
            rs_cw = cwt_ref[h + 1, my]
            rs_ccw = ccwt_ref[h + 1, my]
            for s in range(SEG):
                cw_desc(h, s, rs_cw).wait_recv()
                if h + 1 < N_HOPS:
                    cw_desc(h + 1, s, rs_cw).start()
                ccw_desc(h, s, rs_ccw).wait_recv()
                if h + 1 < N_HOPS:
                    ccw_desc(h + 1, s, rs_ccw).start()

        for h in range(N_HOPS):
            sc_cw = cwt_ref[h, my]
            sc_ccw = ccwt_ref[h, my]
            for s in range(SEG):
                cw_desc(h, s, sc_cw).wait_send()
                ccw_desc(h, s, sc_ccw).wait_send()

        sc = s_ref[0, 0]
        w = w_ref[...]
        for o in range(N_DEV):
            acc = jnp.dot(xg_ref[o], w, preferred_element_type=jnp.float32)
            y = acc * sc
            out_ref[o * m_per:(o + 1) * m_per, :] = y * jax.nn.sigmoid(y)

    return pl.pallas_call(
        body,
        out_shape=jax.ShapeDtypeStruct((m_tot, n_per), jnp.float32),
        in_specs=[
            pl.BlockSpec(memory_space=pltpu.VMEM),
            pl.BlockSpec(memory_space=pltpu.VMEM),
            pl.BlockSpec(memory_space=pltpu.SMEM),
            pl.BlockSpec(memory_space=pltpu.SMEM),
            pl.BlockSpec(memory_space=pltpu.SMEM),
            pl.BlockSpec(memory_space=pltpu.SMEM),
            pl.BlockSpec(memory_space=pltpu.SMEM),
        ],
        out_specs=pl.BlockSpec(memory_space=pltpu.VMEM),
        scratch_shapes=[
            pltpu.VMEM((N_DEV, m_per, k), x.dtype),
            pltpu.SemaphoreType.DMA,
            pltpu.SemaphoreType.DMA((N_DEV, SEG)),
            pltpu.SemaphoreType.DMA((N_DEV, SEG)),
            pltpu.SemaphoreType.DMA((N_DEV, SEG)),
            pltpu.SemaphoreType.DMA((N_DEV, SEG)),
        ],
        compiler_params=pltpu.CompilerParams(collective_id=0),
    )(x, w_mat, scale, jnp.asarray(_RN), jnp.asarray(_LN),
      jnp.asarray(_CW_T), jnp.asarray(_CCW_T))


# device time: 102884 ns/iter; 2.3743x vs baseline; 1.0439x over previous
import numpy as np

import jax
import jax.numpy as jnp
from jax import lax
from jax.experimental import pallas as pl
from jax.experimental.pallas import tpu as pltpu

N_DEV = 32
N_HOPS = N_DEV - 1
SEG = 4


def _ring_tables():
    def lid(x, y, z):
        return z * 8 + y * 2 + (x if y % 2 == 0 else 1 - x)

    c16 = [(0, 0), (1, 0), (2, 0), (3, 0), (3, 1), (2, 1), (1, 1), (1, 2),
           (2, 2), (3, 2), (3, 3), (2, 3), (1, 3), (0, 3), (0, 2), (0, 1)]
    ham = [lid(0, y, z) for (y, z) in c16] + \
          [lid(1, y, z) for (y, z) in reversed(c16)]
    pos = {l: p for p, l in enumerate(ham)}
    rn = np.array([ham[(pos[l] + 1) % N_DEV] for l in range(N_DEV)], np.int32)
    ln = np.array([ham[(pos[l] - 1) % N_DEV] for l in range(N_DEV)], np.int32)
    cw_t = np.array(
        [[ham[(pos[l] - h) % N_DEV] for l in range(N_DEV)]
         for h in range(N_DEV)], np.int32)
    ccw_t = np.array(
        [[ham[(pos[l] + h) % N_DEV] for l in range(N_DEV)]
         for h in range(N_DEV)], np.int32)
    return rn, ln, cw_t, ccw_t


_RN, _LN, _CW_T, _CCW_T = _ring_tables()


def kernel(x, w_mat, scale_x, scale_w):
    m_per, k = x.shape
    _, n_per = w_mat.shape
    m_tot = N_DEV * m_per
    half = m_per // 2
    seg_k = k // SEG

    scale = (scale_x * scale_w).reshape(1, 1)
    x = x.astype(jnp.float8_e5m2)
    w_mat = w_mat.astype(jnp.float8_e5m2)

    def body(x_ref, w_ref, s_ref, rn_ref, ln_ref, cwt_ref, ccwt_ref,
             out_ref, xg_ref, copy_sem, send_cw, recv_cw, send_ccw, recv_ccw):
        my = lax.axis_index("i")
        rn = rn_ref[my]
        ln = ln_ref[my]

        barrier_sem = pltpu.get_barrier_semaphore()
        for nbr in (ln, rn):
            pl.semaphore_signal(
                barrier_sem, inc=1, device_id=(nbr,),
                device_id_type=pl.DeviceIdType.MESH,
            )
        pl.semaphore_wait(barrier_sem, 2)

        lcopy = pltpu.make_async_copy(x_ref, xg_ref.at[my], copy_sem)
        lcopy.start()
        lcopy.wait()

        def cw_desc(h, s, slot):
            blk = xg_ref.at[slot, pl.ds(0, half), pl.ds(s * seg_k, seg_k)]
            return pltpu.make_async_remote_copy(
                src_ref=blk, dst_ref=blk,
                send_sem=send_cw.at[h, s], recv_sem=recv_cw.at[h, s],
                device_id=(rn,), device_id_type=pl.DeviceIdType.MESH,
            )

        def ccw_desc(h, s, slot):
            blk = xg_ref.at[slot, pl.ds(half, half), pl.ds(s * seg_k, seg_k)]
            return pltpu.make_async_remote_copy(
                src_ref=blk, dst_ref=blk,
                send_sem=send_ccw.at[h, s], recv_sem=recv_ccw.at[h, s],
                device_id=(ln,), device_id_type=pl.DeviceIdType.MESH,
            )

        sc = s_ref[0, 0]
        w = w_ref[...]

        def silu_store(acc, row0, rows):
            y = acc * sc
            out_ref[pl.ds(row0, rows), :] = y * jax.nn.sigmoid(y)

        def half_gemm(slot, row_half):
            a = xg_ref[slot, pl.ds(row_half * half, half), :]
            acc = jnp.dot(a, w, preferred_element_type=jnp.float32)
            silu_store(acc, slot * m_per + row_half * half, half)

        for s in range(SEG):
            cw_desc(0, s, my).start()
            ccw_desc(0, s, my).start()

        silu_store(jnp.dot(x_ref[...], w, preferred_element_type=jnp.float32),
                   my * m_per, m_per)

        for h in range(N_HOPS):
            rs_cw = cwt_ref[h + 1, my]
            rs_ccw = ccwt_ref[h + 1, my]
            for s in range(SEG):
                cw_desc(h, s, rs_cw).wait_recv()
                if h + 1 < N_HOPS:
                    cw_desc(h + 1, s, rs_cw).start()
                ccw_desc(h, s, rs_ccw).wait_recv()
                if h + 1 < N_HOPS:
                    ccw_desc(h + 1, s, rs_ccw).start()
            half_gemm(rs_cw, 0)
            half_gemm(rs_ccw, 1)

        for h in range(N_HOPS):
            sc_cw = cwt_ref[h, my]
            sc_ccw = ccwt_ref[h, my]
            for s in range(SEG):
                cw_desc(h, s, sc_cw).wait_send()
                ccw_desc(h, s, sc_ccw).wait_send()

    return pl.pallas_call(
        body,
        out_shape=jax.ShapeDtypeStruct((m_tot, n_per), jnp.float32),
        in_specs=[
            pl.BlockSpec(memory_space=pltpu.VMEM),
            pl.BlockSpec(memory_space=pltpu.VMEM),
            pl.BlockSpec(memory_space=pltpu.SMEM),
            pl.BlockSpec(memory_space=pltpu.SMEM),
            pl.BlockSpec(memory_space=pltpu.SMEM),
            pl.BlockSpec(memory_space=pltpu.SMEM),
            pl.BlockSpec(memory_space=pltpu.SMEM),
        ],
        out_specs=pl.BlockSpec(memory_space=pltpu.VMEM),
        scratch_shapes=[
            pltpu.VMEM((N_DEV, m_per, k), x.dtype),
            pltpu.SemaphoreType.DMA,
            pltpu.SemaphoreType.DMA((N_DEV, SEG)),
            pltpu.SemaphoreType.DMA((N_DEV, SEG)),
            pltpu.SemaphoreType.DMA((N_DEV, SEG)),
            pltpu.SemaphoreType.DMA((N_DEV, SEG)),
        ],
        compiler_params=pltpu.CompilerParams(collective_id=0),
    )(x, w_mat, scale, jnp.asarray(_RN), jnp.asarray(_LN),
      jnp.asarray(_CW_T), jnp.asarray(_CCW_T))
